# baseline (speedup 1.0000x reference)
import jax
import jax.numpy as jnp
from jax.experimental import pallas as pl


def _bn_apply(h_ref, s_ref, b_ref, o_ref):
    o_ref[...] = h_ref[...] * s_ref[...] + b_ref[...]


def kernel(x, edge_index, edge_weight, params):
    n = x.shape[0] * x.shape[1]
    h = x.reshape(n, x.shape[2])
    src = edge_index[0]
    dst = edge_index[1]
    ew = edge_weight.reshape(-1, 1)
    num_layers = len(params)
    for l in range(num_layers):
        p = params[l]
        e_emb = ew @ p['We'] + p['be']
        msg = jnp.take(h, src, axis=0) + e_emb
        aggr = jax.ops.segment_sum(msg, dst, num_segments=n)
        hid = jnp.maximum(aggr @ p['W1'] + p['b1'], 0.0)
        h2 = hid @ p['W2'] + p['b2']
        mean = jnp.mean(h2, axis=0)
        var = jnp.var(h2, axis=0)
        scale = p['gamma'] / jnp.sqrt(var + 1e-5)
        shift = p['beta'] - mean * scale
        h = pl.pallas_call(
            _bn_apply,
            out_shape=jax.ShapeDtypeStruct(h2.shape, h2.dtype),
        )(h2, scale[None, :], shift[None, :])
        if l < num_layers - 1:
            h = jnp.maximum(h, 0.0)
    return h.reshape(x.shape[0], x.shape[1], -1)


# speedup vs baseline: 2.9324x; 2.9324x over previous
import functools

import jax
import jax.numpy as jnp
from jax import lax
from jax.experimental import pallas as pl
from jax.experimental.pallas import tpu as pltpu
from jax.experimental.pallas import tpu_sc as plsc

NC = 2
NS = 16
NW = NC * NS
KE = 80
PW = 128


def _zero_buf(buf, rows, width):
    def zrow(r, _):
        def zcol(j, _):
            buf[r, pl.ds(j * 16, 16)] = jnp.zeros((16,), jnp.float32)
            return 0
        return lax.fori_loop(0, width // 16, zcol, 0)
    lax.fori_loop(0, rows, zrow, 0)


def _zero_acc(acc, buf, s, rps, tail):
    full = rps // KE

    def z(k, _):
        pltpu.sync_copy(buf, acc.at[pl.ds(s * rps + k * KE, KE)])
        return 0
    lax.fori_loop(0, full, z, 0)
    rem = rps - full * KE
    if rem:
        pltpu.sync_copy(buf.at[pl.ds(0, rem)],
                        acc.at[pl.ds(s * rps + full * KE, rem)])
    if tail:
        @pl.when(s == NS - 1)
        def _():
            pltpu.sync_copy(buf.at[pl.ds(0, tail)], acc.at[pl.ds(NS * rps, tail)])


def _copy_out(acc, out_hbm, c, s, rps, tail):
    pltpu.sync_copy(acc.at[pl.ds(s * rps, rps)],
                    out_hbm.at[c, pl.ds(s * rps, rps)])
    if tail:
        @pl.when(s == NS - 1)
        def _():
            pltpu.sync_copy(acc.at[pl.ds(NS * rps, tail)],
                            out_hbm.at[c, pl.ds(NS * rps, tail)])


def _sc_main_body(n, d, e, h_hbm, src_hbm, dst_hbm, out_hbm,
                  sidx_v, didx_v, rows_v, sem, acc):
    c = lax.axis_index("c")
    s = lax.axis_index("s")
    wid = c * NS + s
    iters = (e // NW) // KE
    rps = (n // NS) & ~7
    tail = n - NS * rps

    _zero_buf(rows_v, KE, d)
    _zero_acc(acc, rows_v, s, rps, tail)
    plsc.subcore_barrier()

    pltpu.sync_copy(src_hbm.at[wid], sidx_v)
    pltpu.sync_copy(dst_hbm.at[wid], didx_v)

    def it(i, _):
        pltpu.async_copy(h_hbm.at[sidx_v.at[i]], rows_v, sem).wait()
        pltpu.sync_copy(rows_v, acc.at[didx_v.at[i]], add=True)
        return 0
    lax.fori_loop(0, iters, it, 0)

    plsc.subcore_barrier()
    _copy_out(acc, out_hbm, c, s, rps, tail)


def _sc_deg_body(n, e, ew2_hbm, dst_hbm, dout_hbm, didx_v, ew_v, dacc):
    c = lax.axis_index("c")
    s = lax.axis_index("s")
    wid = c * NS + s
    epw = e // NW
    iters = epw // KE
    rps = (n // NS) & ~7
    tail = n - NS * rps

    _zero_buf(ew_v, KE, PW)
    _zero_acc(dacc, ew_v, s, rps, tail)
    plsc.subcore_barrier()

    pltpu.sync_copy(dst_hbm.at[wid], didx_v)

    def it(i, _):
        pltpu.sync_copy(ew2_hbm.at[pl.ds(wid * epw + i * KE, KE)], ew_v)
        pltpu.sync_copy(ew_v, dacc.at[didx_v.at[i]], add=True)
        return 0
    lax.fori_loop(0, iters, it, 0)

    plsc.subcore_barrier()
    _copy_out(dacc, dout_hbm, c, s, rps, tail)


def _make_sc_main(n, d, e):
    mesh = plsc.VectorSubcoreMesh(core_axis_name="c", subcore_axis_name="s")
    iters = (e // NW) // KE
    return pl.kernel(
        functools.partial(_sc_main_body, n, d, e),
        out_type=(jax.ShapeDtypeStruct((NC, n, d), jnp.float32),),
        mesh=mesh,
        scratch_types=(
            pltpu.VMEM((iters, KE), jnp.int32),
            pltpu.VMEM((iters, KE), jnp.int32),
            pltpu.VMEM((KE, d), jnp.float32),
            pltpu.SemaphoreType.DMA,
            pltpu.VMEM_SHARED((n, d), jnp.float32),
        ),
    )


def _make_sc_deg(n, e):
    mesh = plsc.VectorSubcoreMesh(core_axis_name="c", subcore_axis_name="s")
    iters = (e // NW) // KE
    return pl.kernel(
        functools.partial(_sc_deg_body, n, e),
        out_type=(jax.ShapeDtypeStruct((NC, n, PW), jnp.float32),),
        mesh=mesh,
        scratch_types=(
            pltpu.VMEM((iters, KE), jnp.int32),
            pltpu.VMEM((KE, PW), jnp.float32),
            pltpu.VMEM_SHARED((n, PW), jnp.float32),
        ),
    )


def _mlp_body(p_ref, sw_ref, we_ref, be_ref, w1_ref, b1_ref, w2_ref, b2_ref,
              h2_ref, stats_ref):
    i = pl.program_id(0)
    pp = p_ref[0] + p_ref[1]
    sw = sw_ref[0] + sw_ref[1]
    sumw = sw[:, 0:1]
    deg = sw[:, 1:2]
    aggr = pp + sumw * we_ref[...] + deg * be_ref[...]
    hid = lax.dot_general(aggr, w1_ref[...], (((1,), (0,)), ((), ())),
                          preferred_element_type=jnp.float32) + b1_ref[...]
    hid = jnp.maximum(hid, 0.0)
    h2 = lax.dot_general(hid, w2_ref[...], (((1,), (0,)), ((), ())),
                         preferred_element_type=jnp.float32) + b2_ref[...]
    h2_ref[...] = h2

    @pl.when(i == 0)
    def _():
        stats_ref[...] = jnp.zeros_like(stats_ref)

    stats_ref[0:1, :] += jnp.sum(h2, axis=0, keepdims=True)
    stats_ref[1:2, :] += jnp.sum(h2 * h2, axis=0, keepdims=True)


def _bn_body(relu, n, h2_ref, stats_ref, g_ref, b_ref, o_ref):
    inv_n = 1.0 / n
    mean = stats_ref[0:1, :] * inv_n
    ex2 = stats_ref[1:2, :] * inv_n
    var = ex2 - mean * mean
    scale = g_ref[...] * lax.rsqrt(var + 1e-5)
    shift = b_ref[...] - mean * scale
    o = h2_ref[...] * scale + shift
    if relu:
        o = jnp.maximum(o, 0.0)
    o_ref[...] = o


def _tc_layer(p, swdeg, params, relu, n, d, blk=2000):
    grid = n // blk
    we = params['We'].reshape(1, d)
    be = params['be'].reshape(1, d)
    b1 = params['b1'].reshape(1, -1)
    b2 = params['b2'].reshape(1, -1)
    dh = params['W1'].shape[1]
    h2, stats = pl.pallas_call(
        _mlp_body,
        grid=(grid,),
        in_specs=[
            pl.BlockSpec((NC, blk, d), lambda i: (0, i, 0)),
            pl.BlockSpec((NC, blk, PW), lambda i: (0, i, 0)),
            pl.BlockSpec((1, d), lambda i: (0, 0)),
            pl.BlockSpec((1, d), lambda i: (0, 0)),
            pl.BlockSpec((d, dh), lambda i: (0, 0)),
            pl.BlockSpec((1, dh), lambda i: (0, 0)),
            pl.BlockSpec((dh, d), lambda i: (0, 0)),
            pl.BlockSpec((1, d), lambda i: (0, 0)),
        ],
        out_specs=[
            pl.BlockSpec((blk, d), lambda i: (i, 0)),
            pl.BlockSpec((8, d), lambda i: (0, 0)),
        ],
        out_shape=[
            jax.ShapeDtypeStruct((n, d), jnp.float32),
            jax.ShapeDtypeStruct((8, d), jnp.float32),
        ],
    )(p, swdeg, we, be, params['W1'], b1, params['W2'], b2)

    return pl.pallas_call(
        functools.partial(_bn_body, relu, n),
        grid=(grid,),
        in_specs=[
            pl.BlockSpec((blk, d), lambda i: (i, 0)),
            pl.BlockSpec((8, d), lambda i: (0, 0)),
            pl.BlockSpec((1, d), lambda i: (0, 0)),
            pl.BlockSpec((1, d), lambda i: (0, 0)),
        ],
        out_specs=pl.BlockSpec((blk, d), lambda i: (i, 0)),
        out_shape=jax.ShapeDtypeStruct((n, d), jnp.float32),
    )(h2, stats, params['gamma'].reshape(1, d), params['beta'].reshape(1, d))


def kernel(x, edge_index, edge_weight, params):
    b, g, d = x.shape
    n = b * g
    e = edge_index.shape[1]
    h = x.reshape(n, d)
    src = edge_index[0].reshape(NW, (e // NW) // KE, KE)
    dst = edge_index[1].reshape(NW, (e // NW) // KE, KE)
    ew2 = jnp.zeros((e, PW), jnp.float32)
    ew2 = ew2.at[:, 0].set(edge_weight)
    ew2 = ew2.at[:, 1].set(1.0)

    sc_main = _make_sc_main(n, d, e)
    sc_deg = _make_sc_deg(n, e)

    (swdeg,) = sc_deg(ew2, dst)
    num_layers = len(params)
    for l in range(num_layers):
        (partial,) = sc_main(h, src, dst)
        h = _tc_layer(partial, swdeg, params[l], relu=(l < num_layers - 1),
                      n=n, d=d)
    return h.reshape(b, g, d)


# speedup vs baseline: 5.6918x; 1.9410x over previous
import functools

import jax
import jax.numpy as jnp
from jax import lax
from jax.experimental import pallas as pl
from jax.experimental.pallas import tpu as pltpu
from jax.experimental.pallas import tpu_sc as plsc

NC = 2
NS = 16
NW = NC * NS
KE = 80


def _zero_buf(buf, rows, width):
    def zrow(r, _):
        def zcol(j, _):
            buf[r, pl.ds(j * 16, 16)] = jnp.zeros((16,), jnp.float32)
            return 0
        return lax.fori_loop(0, width // 16, zcol, 0)
    lax.fori_loop(0, rows, zrow, 0)


def _zero_acc(acc, buf, s, rps, tail):
    full = rps // KE

    def z(k, _):
        pltpu.sync_copy(buf, acc.at[pl.ds(s * rps + k * KE, KE)])
        return 0
    lax.fori_loop(0, full, z, 0)
    rem = rps - full * KE
    if rem:
        pltpu.sync_copy(buf.at[pl.ds(0, rem)],
                        acc.at[pl.ds(s * rps + full * KE, rem)])
    if tail:
        @pl.when(s == NS - 1)
        def _():
            pltpu.sync_copy(buf.at[pl.ds(0, tail)], acc.at[pl.ds(NS * rps, tail)])


def _copy_out(acc, out_hbm, c, s, rps, tail):
    pltpu.sync_copy(acc.at[pl.ds(s * rps, rps)],
                    out_hbm.at[c, pl.ds(s * rps, rps)])
    if tail:
        @pl.when(s == NS - 1)
        def _():
            pltpu.sync_copy(acc.at[pl.ds(NS * rps, tail)],
                            out_hbm.at[c, pl.ds(NS * rps, tail)])


def _sc_main_body(n, d, e, shift, h_hbm, pidx_hbm, out_hbm,
                  pidx_v, sidx_r, didx_r, rows_v, gsem, ssem, acc):
    c = lax.axis_index("c")
    s = lax.axis_index("s")
    wid = c * NS + s
    iters = (e // NW) // KE
    rps = (n // NS) & ~7
    tail = n - NS * rps
    mask = (1 << shift) - 1

    _zero_buf(rows_v.at[0], KE, d)
    _zero_acc(acc, rows_v.at[0], s, rps, tail)
    plsc.subcore_barrier()

    pltpu.sync_copy(pidx_hbm.at[wid], pidx_v)

    def unpack(i, slot):
        def u(k, _):
            v = pidx_v[i, pl.ds(k * 16, 16)]
            sidx_r[slot, pl.ds(k * 16, 16)] = lax.shift_right_logical(v, shift)
            didx_r[slot, pl.ds(k * 16, 16)] = lax.bitwise_and(v, mask)
            return 0
        lax.fori_loop(0, KE // 16, u, 0)

    unpack(0, 0)
    pltpu.async_copy(h_hbm.at[sidx_r.at[0]], rows_v.at[0], gsem.at[0])

    def it(i, _):
        b = lax.rem(i, 2)
        nb = 1 - b
        pltpu.make_async_copy(h_hbm.at[sidx_r.at[b]], rows_v.at[b],
                              gsem.at[b]).wait()

        @pl.when(i + 1 < iters)
        def _():
            @pl.when(i >= 1)
            def _():
                pltpu.make_async_copy(
                    rows_v.at[nb], acc.at[didx_r.at[nb]], ssem.at[nb]
                ).wait()
            unpack(i + 1, nb)
            pltpu.async_copy(h_hbm.at[sidx_r.at[nb]], rows_v.at[nb],
                             gsem.at[nb])

        pltpu.async_copy(rows_v.at[b], acc.at[didx_r.at[b]], ssem.at[b],
                         add=True)
        return 0
    lax.fori_loop(0, iters, it, 0)

    lb = (iters - 1) % 2
    pltpu.make_async_copy(rows_v.at[lb], acc.at[didx_r.at[lb]],
                          ssem.at[lb]).wait()
    if iters >= 2:
        pltpu.make_async_copy(rows_v.at[1 - lb], acc.at[didx_r.at[1 - lb]],
                              ssem.at[1 - lb]).wait()

    plsc.subcore_barrier()
    _copy_out(acc, out_hbm, c, s, rps, tail)


def _sc_deg_body(n, d, e, ew2_hbm, dst_hbm, dout_hbm,
                 didx_v, ew_v, gsem, ssem, dacc):
    c = lax.axis_index("c")
    s = lax.axis_index("s")
    wid = c * NS + s
    epw = e // NW
    iters = epw // KE
    rps = (n // NS) & ~7
    tail = n - NS * rps

    _zero_buf(ew_v.at[0], KE, d)
    _zero_acc(dacc, ew_v.at[0], s, rps, tail)
    plsc.subcore_barrier()

    pltpu.sync_copy(dst_hbm.at[wid], didx_v)

    def load_chunk(i, b, issue):
        src = ew2_hbm.at[pl.ds(wid * epw + i * KE, KE)]
        if issue:
            pltpu.async_copy(src, ew_v.at[b], gsem.at[b])
        else:
            pltpu.make_async_copy(src, ew_v.at[b], gsem.at[b]).wait()

    load_chunk(0, 0, issue=True)

    def it(i, _):
        b = lax.rem(i, 2)
        nb = 1 - b
        load_chunk(i, b, issue=False)

        @pl.when(i + 1 < iters)
        def _():
            @pl.when(i >= 1)
            def _():
                pltpu.make_async_copy(
                    ew_v.at[nb], dacc.at[didx_v.at[i - 1]], ssem.at[nb]
                ).wait()
            load_chunk(i + 1, nb, issue=True)

        pltpu.async_copy(ew_v.at[b], dacc.at[didx_v.at[i]], ssem.at[b],
                         add=True)
        return 0
    lax.fori_loop(0, iters, it, 0)

    lb = (iters - 1) % 2
    pltpu.make_async_copy(ew_v.at[lb], dacc.at[didx_v.at[iters - 1]],
                          ssem.at[lb]).wait()
    if iters >= 2:
        pltpu.make_async_copy(ew_v.at[1 - lb], dacc.at[didx_v.at[iters - 2]],
                              ssem.at[1 - lb]).wait()

    plsc.subcore_barrier()
    _copy_out(dacc, dout_hbm, c, s, rps, tail)


def _make_sc_main(n, d, e, shift):
    mesh = plsc.VectorSubcoreMesh(core_axis_name="c", subcore_axis_name="s")
    iters = (e // NW) // KE
    return pl.kernel(
        functools.partial(_sc_main_body, n, d, e, shift),
        out_type=(jax.ShapeDtypeStruct((NC, n, d), jnp.float32),),
        mesh=mesh,
        scratch_types=(
            pltpu.VMEM((iters, KE), jnp.int32),
            pltpu.VMEM((2, KE), jnp.int32),
            pltpu.VMEM((2, KE), jnp.int32),
            pltpu.VMEM((2, KE, d), jnp.float32),
            pltpu.SemaphoreType.DMA((2,)),
            pltpu.SemaphoreType.DMA((2,)),
            pltpu.VMEM_SHARED((n, d), jnp.float32),
        ),
    )


def _make_sc_deg(n, d, e):
    mesh = plsc.VectorSubcoreMesh(core_axis_name="c", subcore_axis_name="s")
    iters = (e // NW) // KE
    return pl.kernel(
        functools.partial(_sc_deg_body, n, d, e),
        out_type=(jax.ShapeDtypeStruct((NC, n, d), jnp.float32),),
        mesh=mesh,
        scratch_types=(
            pltpu.VMEM((iters, KE), jnp.int32),
            pltpu.VMEM((2, KE, d), jnp.float32),
            pltpu.SemaphoreType.DMA((2,)),
            pltpu.SemaphoreType.DMA((2,)),
            pltpu.VMEM_SHARED((n, d), jnp.float32),
        ),
    )


def _ew2_body(ew_ref, o_ref):
    col = lax.broadcasted_iota(jnp.int32, (1, o_ref.shape[1]), 1)
    w = ew_ref[...]
    o_ref[...] = jnp.where(col == 0, w, 0.0) + jnp.where(col == 1, 1.0, 0.0)


def _build_ew2(ew, e, d, blk=8000):
    return pl.pallas_call(
        _ew2_body,
        grid=(e // blk,),
        in_specs=[pl.BlockSpec((blk, 1), lambda i: (i, 0))],
        out_specs=pl.BlockSpec((blk, d), lambda i: (i, 0)),
        out_shape=jax.ShapeDtypeStruct((e, d), jnp.float32),
    )(ew.reshape(e, 1))


def _mlp_body(relu, n, blk, p_ref, sw_ref, we_ref, be_ref, w1_ref, b1_ref,
              w2_ref, b2_ref, g_ref, bb_ref, o_ref, h2_scr, stats_scr):
    ph = pl.program_id(0)
    i = pl.program_id(1)

    @pl.when(ph == 0)
    def _():
        pp = p_ref[0] + p_ref[1]
        sw = sw_ref[0] + sw_ref[1]
        sumw = sw[:, 0:1]
        deg = sw[:, 1:2]
        aggr = pp + sumw * we_ref[...] + deg * be_ref[...]
        hid = lax.dot_general(aggr, w1_ref[...], (((1,), (0,)), ((), ())),
                              preferred_element_type=jnp.float32) + b1_ref[...]
        hid = jnp.maximum(hid, 0.0)
        h2 = lax.dot_general(hid, w2_ref[...], (((1,), (0,)), ((), ())),
                             preferred_element_type=jnp.float32) + b2_ref[...]
        h2_scr[pl.ds(i * blk, blk), :] = h2

        @pl.when(i == 0)
        def _():
            stats_scr[...] = jnp.zeros_like(stats_scr)

        stats_scr[0:1, :] += jnp.sum(h2, axis=0, keepdims=True)
        stats_scr[1:2, :] += jnp.sum(h2 * h2, axis=0, keepdims=True)

    @pl.when(ph == 1)
    def _():
        inv_n = 1.0 / n
        mean = stats_scr[0:1, :] * inv_n
        ex2 = stats_scr[1:2, :] * inv_n
        var = ex2 - mean * mean
        scale = g_ref[...] * lax.rsqrt(var + 1e-5)
        shift = bb_ref[...] - mean * scale
        o = h2_scr[pl.ds(i * blk, blk), :] * scale + shift
        if relu:
            o = jnp.maximum(o, 0.0)
        o_ref[...] = o


def _tc_layer(p, swdeg, params, relu, n, d, blk=2000):
    we = params['We'].reshape(1, d)
    be = params['be'].reshape(1, d)
    b1 = params['b1'].reshape(1, -1)
    b2 = params['b2'].reshape(1, -1)
    dh = params['W1'].shape[1]
    return pl.pallas_call(
        functools.partial(_mlp_body, relu, n, blk),
        grid=(2, n // blk),
        in_specs=[
            pl.BlockSpec((NC, blk, d), lambda p, i: (0, i, 0)),
            pl.BlockSpec((NC, blk, d), lambda p, i: (0, i, 0)),
            pl.BlockSpec((1, d), lambda p, i: (0, 0)),
            pl.BlockSpec((1, d), lambda p, i: (0, 0)),
            pl.BlockSpec((d, dh), lambda p, i: (0, 0)),
            pl.BlockSpec((1, dh), lambda p, i: (0, 0)),
            pl.BlockSpec((dh, d), lambda p, i: (0, 0)),
            pl.BlockSpec((1, d), lambda p, i: (0, 0)),
            pl.BlockSpec((1, d), lambda p, i: (0, 0)),
            pl.BlockSpec((1, d), lambda p, i: (0, 0)),
        ],
        out_specs=pl.BlockSpec((blk, d), lambda p, i: (i, 0)),
        out_shape=jax.ShapeDtypeStruct((n, d), jnp.float32),
        scratch_shapes=[
            pltpu.VMEM((n, d), jnp.float32),
            pltpu.VMEM((8, d), jnp.float32),
        ],
    )(p, swdeg, we, be, params['W1'], b1, params['W2'], b2,
      params['gamma'].reshape(1, d), params['beta'].reshape(1, d))


def kernel(x, edge_index, edge_weight, params):
    b, g, d = x.shape
    n = b * g
    e = edge_index.shape[1]
    h = x.reshape(n, d)
    shift = max(1, (n - 1).bit_length())
    pidx = ((edge_index[0] << shift) | edge_index[1]).reshape(
        NW, (e // NW) // KE, KE)
    dst = edge_index[1].reshape(NW, (e // NW) // KE, KE)

    sc_main = _make_sc_main(n, d, e, shift)
    sc_deg = _make_sc_deg(n, d, e)

    (partial,) = sc_main(h, pidx)
    ew2 = _build_ew2(edge_weight, e, d)
    tok = (partial[0, 0, 0] * 0.0).astype(jnp.int32)
    (swdeg,) = sc_deg(ew2, dst + tok)

    num_layers = len(params)
    for l in range(num_layers):
        if l > 0:
            (partial,) = sc_main(h, pidx)
        h = _tc_layer(partial, swdeg, params[l],
                      relu=(l < num_layers - 1), n=n, d=d)
    return h.reshape(b, g, d)


# speedup vs baseline: 6.4967x; 1.1414x over previous
import functools

import jax
import jax.numpy as jnp
from jax import lax
from jax.experimental import pallas as pl
from jax.experimental.pallas import tpu as pltpu
from jax.experimental.pallas import tpu_sc as plsc

NC = 2
NS = 16
NW = NC * NS
KE = 80


def _zero_buf(buf, rows, width):
    def zrow(r, _):
        def zcol(j, _):
            buf[r, pl.ds(j * 16, 16)] = jnp.zeros((16,), jnp.float32)
            return 0
        return lax.fori_loop(0, width // 16, zcol, 0)
    lax.fori_loop(0, rows, zrow, 0)


def _zero_acc(acc, buf, s, rps, tail):
    full = rps // KE

    def z(k, _):
        pltpu.sync_copy(buf, acc.at[pl.ds(s * rps + k * KE, KE)])
        return 0
    lax.fori_loop(0, full, z, 0)
    rem = rps - full * KE
    if rem:
        pltpu.sync_copy(buf.at[pl.ds(0, rem)],
                        acc.at[pl.ds(s * rps + full * KE, rem)])
    if tail:
        @pl.when(s == NS - 1)
        def _():
            pltpu.sync_copy(buf.at[pl.ds(0, tail)], acc.at[pl.ds(NS * rps, tail)])


def _copy_out(acc, out_hbm, c, s, rps, tail):
    pltpu.sync_copy(acc.at[pl.ds(s * rps, rps)],
                    out_hbm.at[c, pl.ds(s * rps, rps)])
    if tail:
        @pl.when(s == NS - 1)
        def _():
            pltpu.sync_copy(acc.at[pl.ds(NS * rps, tail)],
                            out_hbm.at[c, pl.ds(NS * rps, tail)])


def _sc_main_body(n, d, e, h_hbm, src_hbm, dst_hbm, out_hbm,
                  sidx_r, didx_r, rows_v, isem, gsem, ssem, acc):
    c = lax.axis_index("c")
    s = lax.axis_index("s")
    wid = c * NS + s
    epw = e // NW
    iters = epw // KE
    rps = (n // NS) & ~7
    tail = n - NS * rps

    _zero_buf(rows_v.at[0], KE, d)
    _zero_acc(acc, rows_v.at[0], s, rps, tail)
    plsc.subcore_barrier()

    def idx_pair(j, slot):
        base = wid * epw + j * KE
        return (
            (src_hbm.at[pl.ds(base, KE)], sidx_r.at[slot]),
            (dst_hbm.at[pl.ds(base, KE)], didx_r.at[slot]),
        )

    def idxload(j, slot):
        for srcd, dstd in idx_pair(j, slot):
            pltpu.async_copy(srcd, dstd, isem.at[slot])

    def idxwait(j, slot):
        for srcd, dstd in idx_pair(j, slot):
            pltpu.make_async_copy(srcd, dstd, isem.at[slot]).wait()

    def gather(j, rb, issue):
        if issue:
            pltpu.async_copy(h_hbm.at[sidx_r.at[lax.rem(j, 4)]],
                             rows_v.at[rb], gsem.at[rb])
        else:
            pltpu.make_async_copy(h_hbm.at[sidx_r.at[lax.rem(j, 4)]],
                                  rows_v.at[rb], gsem.at[rb]).wait()

    def scatter(j, rb, issue):
        if issue:
            pltpu.async_copy(rows_v.at[rb], acc.at[didx_r.at[lax.rem(j, 4)]],
                             ssem.at[rb], add=True)
        else:
            pltpu.make_async_copy(rows_v.at[rb],
                                  acc.at[didx_r.at[lax.rem(j, 4)]],
                                  ssem.at[rb]).wait()

    idxload(0, 0)
    idxload(1, 1)
    idxload(2, 2)
    idxwait(0, 0)
    gather(0, 0, issue=True)
    idxwait(1, 1)
    gather(1, 1, issue=True)

    def it(i, _):
        b = lax.rem(i, 3)
        gather(i, b, issue=False)

        @pl.when(i >= 1)
        def _():
            scatter(i - 1, lax.rem(i - 1, 3), issue=False)

        @pl.when(i + 3 < iters)
        def _():
            idxload(i + 3, lax.rem(i + 3, 4))

        @pl.when(i + 2 < iters)
        def _():
            idxwait(i + 2, lax.rem(i + 2, 4))
            gather(i + 2, lax.rem(i + 2, 3), issue=True)

        scatter(i, b, issue=True)
        return 0
    lax.fori_loop(0, iters, it, 0)

    scatter(iters - 1, (iters - 1) % 3, issue=False)

    plsc.subcore_barrier()
    _copy_out(acc, out_hbm, c, s, rps, tail)


def _sc_deg_body(n, d, e, ew2_hbm, dst_hbm, dout_hbm,
                 didx_v, ew_v, gsem, ssem, dacc):
    c = lax.axis_index("c")
    s = lax.axis_index("s")
    wid = c * NS + s
    epw = e // NW
    iters = epw // KE
    rps = (n // NS) & ~7
    tail = n - NS * rps

    _zero_buf(ew_v.at[0], KE, d)
    _zero_acc(dacc, ew_v.at[0], s, rps, tail)
    plsc.subcore_barrier()

    pltpu.sync_copy(dst_hbm.at[wid], didx_v)

    def load_chunk(i, b, issue):
        src = ew2_hbm.at[pl.ds(wid * epw + i * KE, KE)]
        if issue:
            pltpu.async_copy(src, ew_v.at[b], gsem.at[b])
        else:
            pltpu.make_async_copy(src, ew_v.at[b], gsem.at[b]).wait()

    load_chunk(0, 0, issue=True)

    def it(i, _):
        b = lax.rem(i, 2)
        nb = 1 - b
        load_chunk(i, b, issue=False)

        @pl.when(i + 1 < iters)
        def _():
            @pl.when(i >= 1)
            def _():
                pltpu.make_async_copy(
                    ew_v.at[nb], dacc.at[didx_v.at[i - 1]], ssem.at[nb]
                ).wait()
            load_chunk(i + 1, nb, issue=True)

        pltpu.async_copy(ew_v.at[b], dacc.at[didx_v.at[i]], ssem.at[b],
                         add=True)
        return 0
    lax.fori_loop(0, iters, it, 0)

    lb = (iters - 1) % 2
    pltpu.make_async_copy(ew_v.at[lb], dacc.at[didx_v.at[iters - 1]],
                          ssem.at[lb]).wait()
    if iters >= 2:
        pltpu.make_async_copy(ew_v.at[1 - lb], dacc.at[didx_v.at[iters - 2]],
                              ssem.at[1 - lb]).wait()

    plsc.subcore_barrier()
    _copy_out(dacc, dout_hbm, c, s, rps, tail)


def _make_sc_main(n, d, e):
    mesh = plsc.VectorSubcoreMesh(core_axis_name="c", subcore_axis_name="s")
    return pl.kernel(
        functools.partial(_sc_main_body, n, d, e),
        out_type=(jax.ShapeDtypeStruct((NC, n, d), jnp.float32),),
        mesh=mesh,
        scratch_types=(
            pltpu.VMEM((4, KE), jnp.int32),
            pltpu.VMEM((4, KE), jnp.int32),
            pltpu.VMEM((3, KE, d), jnp.float32),
            pltpu.SemaphoreType.DMA((4,)),
            pltpu.SemaphoreType.DMA((3,)),
            pltpu.SemaphoreType.DMA((3,)),
            pltpu.VMEM_SHARED((n, d), jnp.float32),
        ),
    )


def _make_sc_deg(n, d, e):
    mesh = plsc.VectorSubcoreMesh(core_axis_name="c", subcore_axis_name="s")
    iters = (e // NW) // KE
    return pl.kernel(
        functools.partial(_sc_deg_body, n, d, e),
        out_type=(jax.ShapeDtypeStruct((NC, n, d), jnp.float32),),
        mesh=mesh,
        scratch_types=(
            pltpu.VMEM((iters, KE), jnp.int32),
            pltpu.VMEM((2, KE, d), jnp.float32),
            pltpu.SemaphoreType.DMA((2,)),
            pltpu.SemaphoreType.DMA((2,)),
            pltpu.VMEM_SHARED((n, d), jnp.float32),
        ),
    )


def _ew2_body(ew_ref, o_ref):
    col = lax.broadcasted_iota(jnp.int32, (1, o_ref.shape[1]), 1)
    w = ew_ref[...]
    o_ref[...] = jnp.where(col == 0, w, 0.0) + jnp.where(col == 1, 1.0, 0.0)


def _build_ew2(ew, e, d, blk=8000):
    return pl.pallas_call(
        _ew2_body,
        grid=(e // blk,),
        in_specs=[pl.BlockSpec((blk, 1), lambda i: (i, 0))],
        out_specs=pl.BlockSpec((blk, d), lambda i: (i, 0)),
        out_shape=jax.ShapeDtypeStruct((e, d), jnp.float32),
    )(ew.reshape(e, 1))


def _mlp_body(relu, n, blk, p_ref, sw_ref, we_ref, be_ref, w1_ref, b1_ref,
              w2_ref, b2_ref, g_ref, bb_ref, o_ref, h2_scr, stats_scr):
    ph = pl.program_id(0)
    i = pl.program_id(1)

    @pl.when(ph == 0)
    def _():
        pp = p_ref[0] + p_ref[1]
        sw = sw_ref[0] + sw_ref[1]
        sumw = sw[:, 0:1]
        deg = sw[:, 1:2]
        aggr = pp + sumw * we_ref[...] + deg * be_ref[...]
        hid = lax.dot_general(aggr, w1_ref[...], (((1,), (0,)), ((), ())),
                              preferred_element_type=jnp.float32) + b1_ref[...]
        hid = jnp.maximum(hid, 0.0)
        h2 = lax.dot_general(hid, w2_ref[...], (((1,), (0,)), ((), ())),
                             preferred_element_type=jnp.float32) + b2_ref[...]
        h2_scr[pl.ds(i * blk, blk), :] = h2

        @pl.when(i == 0)
        def _():
            stats_scr[...] = jnp.zeros_like(stats_scr)

        stats_scr[0:1, :] += jnp.sum(h2, axis=0, keepdims=True)
        stats_scr[1:2, :] += jnp.sum(h2 * h2, axis=0, keepdims=True)

    @pl.when(ph == 1)
    def _():
        inv_n = 1.0 / n
        mean = stats_scr[0:1, :] * inv_n
        ex2 = stats_scr[1:2, :] * inv_n
        var = ex2 - mean * mean
        scale = g_ref[...] * lax.rsqrt(var + 1e-5)
        shift = bb_ref[...] - mean * scale
        o = h2_scr[pl.ds(i * blk, blk), :] * scale + shift
        if relu:
            o = jnp.maximum(o, 0.0)
        o_ref[...] = o


def _tc_layer(p, swdeg, params, relu, n, d, blk=2000):
    we = params['We'].reshape(1, d)
    be = params['be'].reshape(1, d)
    b1 = params['b1'].reshape(1, -1)
    b2 = params['b2'].reshape(1, -1)
    dh = params['W1'].shape[1]
    return pl.pallas_call(
        functools.partial(_mlp_body, relu, n, blk),
        grid=(2, n // blk),
        in_specs=[
            pl.BlockSpec((NC, blk, d), lambda p, i: (0, i, 0)),
            pl.BlockSpec((NC, blk, d), lambda p, i: (0, i, 0)),
            pl.BlockSpec((1, d), lambda p, i: (0, 0)),
            pl.BlockSpec((1, d), lambda p, i: (0, 0)),
            pl.BlockSpec((d, dh), lambda p, i: (0, 0)),
            pl.BlockSpec((1, dh), lambda p, i: (0, 0)),
            pl.BlockSpec((dh, d), lambda p, i: (0, 0)),
            pl.BlockSpec((1, d), lambda p, i: (0, 0)),
            pl.BlockSpec((1, d), lambda p, i: (0, 0)),
            pl.BlockSpec((1, d), lambda p, i: (0, 0)),
        ],
        out_specs=pl.BlockSpec((blk, d), lambda p, i: (i, 0)),
        out_shape=jax.ShapeDtypeStruct((n, d), jnp.float32),
        scratch_shapes=[
            pltpu.VMEM((n, d), jnp.float32),
            pltpu.VMEM((8, d), jnp.float32),
        ],
    )(p, swdeg, we, be, params['W1'], b1, params['W2'], b2,
      params['gamma'].reshape(1, d), params['beta'].reshape(1, d))


def kernel(x, edge_index, edge_weight, params):
    b, g, d = x.shape
    n = b * g
    e = edge_index.shape[1]
    h = x.reshape(n, d)
    src1 = edge_index[0]
    dst1 = edge_index[1]
    dst = edge_index[1].reshape(NW, (e // NW) // KE, KE)

    sc_main = _make_sc_main(n, d, e)
    sc_deg = _make_sc_deg(n, d, e)

    (partial,) = sc_main(h, src1, dst1)
    ew2 = _build_ew2(edge_weight, e, d)
    tok = (partial[0, 0, 0] * 0.0).astype(jnp.int32)
    (swdeg,) = sc_deg(ew2, dst + tok)

    num_layers = len(params)
    for l in range(num_layers):
        if l > 0:
            (partial,) = sc_main(h, src1, dst1)
        h = _tc_layer(partial, swdeg, params[l],
                      relu=(l < num_layers - 1), n=n, d=d)
    return h.reshape(b, g, d)


# speedup vs baseline: 11.3528x; 1.7475x over previous
import functools

import jax
import jax.numpy as jnp
from jax import lax
from jax.experimental import pallas as pl
from jax.experimental.pallas import tpu as pltpu
from jax.experimental.pallas import tpu_sc as plsc

NC = 2
NS = 16
NW = NC * NS
KE = 80


def _zero_buf(buf, rows, width):
    def zrow(r, _):
        def zcol(j, _):
            buf[r, pl.ds(j * 16, 16)] = jnp.zeros((16,), jnp.float32)
            return 0
        return lax.fori_loop(0, width // 16, zcol, 0)
    lax.fori_loop(0, rows, zrow, 0)


def _zero_acc(acc, buf, s, rps, tail):
    full = rps // KE

    def z(k, _):
        pltpu.sync_copy(buf, acc.at[pl.ds(s * rps + k * KE, KE)])
        return 0
    lax.fori_loop(0, full, z, 0)
    rem = rps - full * KE
    if rem:
        pltpu.sync_copy(buf.at[pl.ds(0, rem)],
                        acc.at[pl.ds(s * rps + full * KE, rem)])
    if tail:
        @pl.when(s == NS - 1)
        def _():
            pltpu.sync_copy(buf.at[pl.ds(0, tail)], acc.at[pl.ds(NS * rps, tail)])


def _copy_out(acc, out_hbm, c, s, rps, tail):
    pltpu.sync_copy(acc.at[pl.ds(s * rps, rps)],
                    out_hbm.at[c, pl.ds(s * rps, rps)])
    if tail:
        @pl.when(s == NS - 1)
        def _():
            pltpu.sync_copy(acc.at[pl.ds(NS * rps, tail)],
                            out_hbm.at[c, pl.ds(NS * rps, tail)])


def _sc_main_body(n, d, e, h_hbm, ei_hbm, out_hbm,
                  sidx_r, didx_r, rows_v, isem, gsem, ssem, acc):
    c = lax.axis_index("c")
    s = lax.axis_index("s")
    wid = c * NS + s
    epw = e // NW
    iters = epw // KE
    rps = (n // NS) & ~7
    tail = n - NS * rps

    _zero_buf(rows_v.at[0], KE, d)
    _zero_acc(acc, rows_v.at[0], s, rps, tail)
    plsc.subcore_barrier()

    def idx_pair(j, slot):
        base = wid * epw + j * KE
        return (
            (ei_hbm.at[pl.ds(base, KE)], sidx_r.at[slot]),
            (ei_hbm.at[pl.ds(e + base, KE)], didx_r.at[slot]),
        )

    def idxload(j, slot):
        for srcd, dstd in idx_pair(j, slot):
            pltpu.async_copy(srcd, dstd, isem.at[slot])

    def idxwait(j, slot):
        for srcd, dstd in idx_pair(j, slot):
            pltpu.make_async_copy(srcd, dstd, isem.at[slot]).wait()

    def gather(j, rb, issue):
        if issue:
            pltpu.async_copy(h_hbm.at[sidx_r.at[lax.rem(j, 4)]],
                             rows_v.at[rb], gsem.at[rb])
        else:
            pltpu.make_async_copy(h_hbm.at[sidx_r.at[lax.rem(j, 4)]],
                                  rows_v.at[rb], gsem.at[rb]).wait()

    def scatter(j, rb, issue):
        if issue:
            pltpu.async_copy(rows_v.at[rb], acc.at[didx_r.at[lax.rem(j, 4)]],
                             ssem.at[rb], add=True)
        else:
            pltpu.make_async_copy(rows_v.at[rb],
                                  acc.at[didx_r.at[lax.rem(j, 4)]],
                                  ssem.at[rb]).wait()

    idxload(0, 0)
    idxload(1, 1)
    idxload(2, 2)
    idxwait(0, 0)
    gather(0, 0, issue=True)
    idxwait(1, 1)
    gather(1, 1, issue=True)

    def it(i, _):
        b = lax.rem(i, 3)
        gather(i, b, issue=False)

        @pl.when(i >= 1)
        def _():
            scatter(i - 1, lax.rem(i - 1, 3), issue=False)

        @pl.when(i + 3 < iters)
        def _():
            idxload(i + 3, lax.rem(i + 3, 4))

        @pl.when(i + 2 < iters)
        def _():
            idxwait(i + 2, lax.rem(i + 2, 4))
            gather(i + 2, lax.rem(i + 2, 3), issue=True)

        scatter(i, b, issue=True)
        return 0
    lax.fori_loop(0, iters, it, 0)

    scatter(iters - 1, (iters - 1) % 3, issue=False)

    plsc.subcore_barrier()
    _copy_out(acc, out_hbm, c, s, rps, tail)


def _sc_deg_body(n, d, e, ei_hbm, ew_hbm, dout_hbm,
                 didx_r, ewc_r, ew_v, isem, ssem, dacc):
    c = lax.axis_index("c")
    s = lax.axis_index("s")
    wid = c * NS + s
    epw = e // NW
    iters = epw // KE
    rps = (n // NS) & ~7
    tail = n - NS * rps

    _zero_buf(ew_v.at[0], KE, d)
    _zero_acc(dacc, ew_v.at[0], s, rps, tail)
    _zero_buf(ew_v.at[1], KE, d)
    _zero_buf(ew_v.at[2], KE, d)
    plsc.subcore_barrier()

    def chunk_pair(j, slot):
        base = wid * epw + j * KE
        return (
            (ei_hbm.at[pl.ds(e + base, KE)], didx_r.at[slot]),
            (ew_hbm.at[pl.ds(base, KE)], ewc_r.at[slot]),
        )

    def chunkload(j, slot):
        for srcd, dstd in chunk_pair(j, slot):
            pltpu.async_copy(srcd, dstd, isem.at[slot])

    def chunkwait(j, slot):
        for srcd, dstd in chunk_pair(j, slot):
            pltpu.make_async_copy(srcd, dstd, isem.at[slot]).wait()

    iota16 = lax.iota(jnp.int32, 16)
    one16 = jnp.where(iota16 == 1, 1.0, 0.0)

    def fill(slot, rb):
        def f16(k, _):
            w16 = ewc_r[slot, pl.ds(k * 16, 16)]
            for j in range(16):
                ew_v[rb, k * 16 + j, pl.ds(0, 16)] = jnp.where(
                    iota16 == 0, w16[j], one16)
            return 0
        lax.fori_loop(0, KE // 16, f16, 0)

    def scatter(j, rb, issue):
        if issue:
            pltpu.async_copy(ew_v.at[rb], dacc.at[didx_r.at[lax.rem(j, 5)]],
                             ssem.at[rb], add=True)
        else:
            pltpu.make_async_copy(ew_v.at[rb],
                                  dacc.at[didx_r.at[lax.rem(j, 5)]],
                                  ssem.at[rb]).wait()

    chunkload(0, 0)
    chunkload(1, 1)
    chunkload(2, 2)

    def it(i, _):
        @pl.when(i >= 2)
        def _():
            scatter(i - 2, lax.rem(i - 2, 3), issue=False)

        @pl.when(i + 3 < iters)
        def _():
            chunkload(i + 3, lax.rem(i + 3, 5))

        chunkwait(i, lax.rem(i, 5))
        fill(lax.rem(i, 5), lax.rem(i, 3))
        scatter(i, lax.rem(i, 3), issue=True)
        return 0
    lax.fori_loop(0, iters, it, 0)

    scatter(iters - 2, (iters - 2) % 3, issue=False)
    scatter(iters - 1, (iters - 1) % 3, issue=False)

    plsc.subcore_barrier()
    _copy_out(dacc, dout_hbm, c, s, rps, tail)


def _make_sc_main(n, d, e):
    mesh = plsc.VectorSubcoreMesh(core_axis_name="c", subcore_axis_name="s")
    return pl.kernel(
        functools.partial(_sc_main_body, n, d, e),
        out_type=(jax.ShapeDtypeStruct((NC, n, d), jnp.float32),),
        mesh=mesh,
        scratch_types=(
            pltpu.VMEM((4, KE), jnp.int32),
            pltpu.VMEM((4, KE), jnp.int32),
            pltpu.VMEM((3, KE, d), jnp.float32),
            pltpu.SemaphoreType.DMA((4,)),
            pltpu.SemaphoreType.DMA((3,)),
            pltpu.SemaphoreType.DMA((3,)),
            pltpu.VMEM_SHARED((n, d), jnp.float32),
        ),
    )


def _make_sc_deg(n, d, e):
    mesh = plsc.VectorSubcoreMesh(core_axis_name="c", subcore_axis_name="s")
    return pl.kernel(
        functools.partial(_sc_deg_body, n, d, e),
        out_type=(jax.ShapeDtypeStruct((NC, n, d), jnp.float32),),
        mesh=mesh,
        scratch_types=(
            pltpu.VMEM((5, KE), jnp.int32),
            pltpu.VMEM((5, KE), jnp.float32),
            pltpu.VMEM((3, KE, d), jnp.float32),
            pltpu.SemaphoreType.DMA((5,)),
            pltpu.SemaphoreType.DMA((3,)),
            pltpu.VMEM_SHARED((n, d), jnp.float32),
        ),
    )


def _ew2_body(ew_ref, o_ref):
    col = lax.broadcasted_iota(jnp.int32, (1, o_ref.shape[1]), 1)
    w = ew_ref[...]
    o_ref[...] = jnp.where(col == 0, w, 0.0) + jnp.where(col == 1, 1.0, 0.0)


def _build_ew2(ew, e, d, blk=8000):
    return pl.pallas_call(
        _ew2_body,
        grid=(e // blk,),
        in_specs=[pl.BlockSpec((blk, 1), lambda i: (i, 0))],
        out_specs=pl.BlockSpec((blk, d), lambda i: (i, 0)),
        out_shape=jax.ShapeDtypeStruct((e, d), jnp.float32),
    )(ew.reshape(e, 1))


def _mlp_body(relu, n, blk, p_ref, sw_ref, we_ref, be_ref, w1_ref, b1_ref,
              w2_ref, b2_ref, g_ref, bb_ref, o_ref, h2_scr, stats_scr):
    ph = pl.program_id(0)
    i = pl.program_id(1)

    @pl.when(ph == 0)
    def _():
        pp = p_ref[0] + p_ref[1]
        sw = sw_ref[0] + sw_ref[1]
        sumw = sw[:, 0:1]
        deg = sw[:, 1:2]
        aggr = pp + sumw * we_ref[...] + deg * be_ref[...]
        hid = lax.dot_general(aggr, w1_ref[...], (((1,), (0,)), ((), ())),
                              preferred_element_type=jnp.float32) + b1_ref[...]
        hid = jnp.maximum(hid, 0.0)
        h2 = lax.dot_general(hid, w2_ref[...], (((1,), (0,)), ((), ())),
                             preferred_element_type=jnp.float32) + b2_ref[...]
        h2_scr[pl.ds(i * blk, blk), :] = h2

        @pl.when(i == 0)
        def _():
            stats_scr[...] = jnp.zeros_like(stats_scr)

        stats_scr[0:1, :] += jnp.sum(h2, axis=0, keepdims=True)
        stats_scr[1:2, :] += jnp.sum(h2 * h2, axis=0, keepdims=True)

    @pl.when(ph == 1)
    def _():
        inv_n = 1.0 / n
        mean = stats_scr[0:1, :] * inv_n
        ex2 = stats_scr[1:2, :] * inv_n
        var = ex2 - mean * mean
        scale = g_ref[...] * lax.rsqrt(var + 1e-5)
        shift = bb_ref[...] - mean * scale
        o = h2_scr[pl.ds(i * blk, blk), :] * scale + shift
        if relu:
            o = jnp.maximum(o, 0.0)
        o_ref[...] = o


def _tc_layer(p, swdeg, params, relu, n, d, blk=2000):
    we = params['We'].reshape(1, d)
    be = params['be'].reshape(1, d)
    b1 = params['b1'].reshape(1, -1)
    b2 = params['b2'].reshape(1, -1)
    dh = params['W1'].shape[1]
    return pl.pallas_call(
        functools.partial(_mlp_body, relu, n, blk),
        grid=(2, n // blk),
        in_specs=[
            pl.BlockSpec((NC, blk, d), lambda p, i: (0, i, 0)),
            pl.BlockSpec((NC, blk, d), lambda p, i: (0, i, 0)),
            pl.BlockSpec((1, d), lambda p, i: (0, 0)),
            pl.BlockSpec((1, d), lambda p, i: (0, 0)),
            pl.BlockSpec((d, dh), lambda p, i: (0, 0)),
            pl.BlockSpec((1, dh), lambda p, i: (0, 0)),
            pl.BlockSpec((dh, d), lambda p, i: (0, 0)),
            pl.BlockSpec((1, d), lambda p, i: (0, 0)),
            pl.BlockSpec((1, d), lambda p, i: (0, 0)),
            pl.BlockSpec((1, d), lambda p, i: (0, 0)),
        ],
        out_specs=pl.BlockSpec((blk, d), lambda p, i: (i, 0)),
        out_shape=jax.ShapeDtypeStruct((n, d), jnp.float32),
        scratch_shapes=[
            pltpu.VMEM((n, d), jnp.float32),
            pltpu.VMEM((8, d), jnp.float32),
        ],
    )(p, swdeg, we, be, params['W1'], b1, params['W2'], b2,
      params['gamma'].reshape(1, d), params['beta'].reshape(1, d))


def kernel(x, edge_index, edge_weight, params):
    b, g, d = x.shape
    n = b * g
    e = edge_index.shape[1]
    h = x.reshape(n, d)
    ei = edge_index.reshape(2 * e)

    sc_main = _make_sc_main(n, d, e)
    sc_deg = _make_sc_deg(n, d, e)

    (partial,) = sc_main(h, ei)
    tok = (partial[0, 0, 0] * 0.0).astype(jnp.int32)
    (swdeg,) = sc_deg(ei + tok, edge_weight)

    num_layers = len(params)
    for l in range(num_layers):
        if l > 0:
            (partial,) = sc_main(h, ei)
        h = _tc_layer(partial, swdeg, params[l],
                      relu=(l < num_layers - 1), n=n, d=d)
    return h.reshape(b, g, d)


# speedup vs baseline: 11.3629x; 1.0009x over previous
import functools

import jax
import jax.numpy as jnp
from jax import lax
from jax.experimental import pallas as pl
from jax.experimental.pallas import tpu as pltpu
from jax.experimental.pallas import tpu_sc as plsc

NC = 2
NS = 16
NW = NC * NS
KE = 80


def _zero_buf(buf, rows, width):
    def zrow(r, _):
        def zcol(j, _):
            buf[r, pl.ds(j * 16, 16)] = jnp.zeros((16,), jnp.float32)
            return 0
        return lax.fori_loop(0, width // 16, zcol, 0)
    lax.fori_loop(0, rows, zrow, 0)


def _zero_acc(acc, buf, s, rps, tail):
    full = rps // KE

    def z(k, _):
        pltpu.sync_copy(buf, acc.at[pl.ds(s * rps + k * KE, KE)])
        return 0
    lax.fori_loop(0, full, z, 0)
    rem = rps - full * KE
    if rem:
        pltpu.sync_copy(buf.at[pl.ds(0, rem)],
                        acc.at[pl.ds(s * rps + full * KE, rem)])
    if tail:
        @pl.when(s == NS - 1)
        def _():
            pltpu.sync_copy(buf.at[pl.ds(0, tail)], acc.at[pl.ds(NS * rps, tail)])


def _copy_out(acc, out_hbm, c, s, rps, tail):
    pltpu.sync_copy(acc.at[pl.ds(s * rps, rps)],
                    out_hbm.at[c, pl.ds(s * rps, rps)])
    if tail:
        @pl.when(s == NS - 1)
        def _():
            pltpu.sync_copy(acc.at[pl.ds(NS * rps, tail)],
                            out_hbm.at[c, pl.ds(NS * rps, tail)])


def _sc_main_body(n, d, e, h_hbm, ei_hbm, out_hbm,
                  sidx_r, didx_r, rows_v, isem, gsem, ssem, acc):
    c = lax.axis_index("c")
    s = lax.axis_index("s")
    wid = c * NS + s
    epw = e // NW
    iters = epw // KE
    rps = (n // NS) & ~7
    tail = n - NS * rps

    _zero_buf(rows_v.at[0], KE, d)
    _zero_acc(acc, rows_v.at[0], s, rps, tail)
    plsc.subcore_barrier()

    def idx_pair(j, slot):
        base = wid * epw + j * KE
        return (
            (ei_hbm.at[pl.ds(base, KE)], sidx_r.at[slot]),
            (ei_hbm.at[pl.ds(e + base, KE)], didx_r.at[slot]),
        )

    def idxload(j, slot):
        for srcd, dstd in idx_pair(j, slot):
            pltpu.async_copy(srcd, dstd, isem.at[slot])

    def idxwait(j, slot):
        for srcd, dstd in idx_pair(j, slot):
            pltpu.make_async_copy(srcd, dstd, isem.at[slot]).wait()

    def gather(j, rb, issue):
        if issue:
            pltpu.async_copy(h_hbm.at[sidx_r.at[lax.rem(j, 4)]],
                             rows_v.at[rb], gsem.at[rb])
        else:
            pltpu.make_async_copy(h_hbm.at[sidx_r.at[lax.rem(j, 4)]],
                                  rows_v.at[rb], gsem.at[rb]).wait()

    def scatter(j, rb, issue):
        if issue:
            pltpu.async_copy(rows_v.at[rb], acc.at[didx_r.at[lax.rem(j, 4)]],
                             ssem.at[rb], add=True)
        else:
            pltpu.make_async_copy(rows_v.at[rb],
                                  acc.at[didx_r.at[lax.rem(j, 4)]],
                                  ssem.at[rb]).wait()

    idxload(0, 0)
    idxload(1, 1)
    idxload(2, 2)
    idxwait(0, 0)
    gather(0, 0, issue=True)
    idxwait(1, 1)
    gather(1, 1, issue=True)

    def it(i, _):
        b = lax.rem(i, 3)
        gather(i, b, issue=False)

        @pl.when(i >= 1)
        def _():
            scatter(i - 1, lax.rem(i - 1, 3), issue=False)

        @pl.when(i + 3 < iters)
        def _():
            idxload(i + 3, lax.rem(i + 3, 4))

        @pl.when(i + 2 < iters)
        def _():
            idxwait(i + 2, lax.rem(i + 2, 4))
            gather(i + 2, lax.rem(i + 2, 3), issue=True)

        scatter(i, b, issue=True)
        return 0
    lax.fori_loop(0, iters, it, 0)

    scatter(iters - 1, (iters - 1) % 3, issue=False)

    plsc.subcore_barrier()
    _copy_out(acc, out_hbm, c, s, rps, tail)


def _sc_deg_body(n, d, e, ei_hbm, ew_hbm, dout_hbm,
                 didx_r, ewc_r, ew_v, isem, ssem, dacc):
    c = lax.axis_index("c")
    s = lax.axis_index("s")
    wid = c * NS + s
    epw = e // NW
    iters = epw // KE
    rps = (n // NS) & ~7
    tail = n - NS * rps

    _zero_buf(ew_v.at[0], KE, d)
    _zero_acc(dacc, ew_v.at[0], s, rps, tail)
    _zero_buf(ew_v.at[1], KE, d)
    _zero_buf(ew_v.at[2], KE, d)
    plsc.subcore_barrier()

    def chunk_pair(j, slot):
        base = wid * epw + j * KE
        return (
            (ei_hbm.at[pl.ds(e + base, KE)], didx_r.at[slot]),
            (ew_hbm.at[pl.ds(base, KE)], ewc_r.at[slot]),
        )

    def chunkload(j, slot):
        for srcd, dstd in chunk_pair(j, slot):
            pltpu.async_copy(srcd, dstd, isem.at[slot])

    def chunkwait(j, slot):
        for srcd, dstd in chunk_pair(j, slot):
            pltpu.make_async_copy(srcd, dstd, isem.at[slot]).wait()

    iota16 = lax.iota(jnp.int32, 16)
    one16 = jnp.where(iota16 == 1, 1.0, 0.0)

    def fill(slot, rb):
        def f16(k, _):
            w16 = ewc_r[slot, pl.ds(k * 16, 16)]
            for j in range(16):
                ew_v[rb, k * 16 + j, pl.ds(0, 16)] = jnp.where(
                    iota16 == 0, w16[j], one16)
            return 0
        lax.fori_loop(0, KE // 16, f16, 0)

    def scatter(j, rb, issue):
        if issue:
            pltpu.async_copy(ew_v.at[rb], dacc.at[didx_r.at[lax.rem(j, 5)]],
                             ssem.at[rb], add=True)
        else:
            pltpu.make_async_copy(ew_v.at[rb],
                                  dacc.at[didx_r.at[lax.rem(j, 5)]],
                                  ssem.at[rb]).wait()

    chunkload(0, 0)
    chunkload(1, 1)
    chunkload(2, 2)

    def it(i, _):
        @pl.when(i >= 2)
        def _():
            scatter(i - 2, lax.rem(i - 2, 3), issue=False)

        @pl.when(i + 3 < iters)
        def _():
            chunkload(i + 3, lax.rem(i + 3, 5))

        chunkwait(i, lax.rem(i, 5))
        fill(lax.rem(i, 5), lax.rem(i, 3))
        scatter(i, lax.rem(i, 3), issue=True)
        return 0
    lax.fori_loop(0, iters, it, 0)

    scatter(iters - 2, (iters - 2) % 3, issue=False)
    scatter(iters - 1, (iters - 1) % 3, issue=False)

    plsc.subcore_barrier()
    _copy_out(dacc, dout_hbm, c, s, rps, tail)


def _make_sc_main(n, d, e):
    mesh = plsc.VectorSubcoreMesh(core_axis_name="c", subcore_axis_name="s")
    return pl.kernel(
        functools.partial(_sc_main_body, n, d, e),
        out_type=(jax.ShapeDtypeStruct((NC, n, d), jnp.float32),),
        mesh=mesh,
        scratch_types=(
            pltpu.VMEM((4, KE), jnp.int32),
            pltpu.VMEM((4, KE), jnp.int32),
            pltpu.VMEM((3, KE, d), jnp.float32),
            pltpu.SemaphoreType.DMA((4,)),
            pltpu.SemaphoreType.DMA((3,)),
            pltpu.SemaphoreType.DMA((3,)),
            pltpu.VMEM_SHARED((n, d), jnp.float32),
        ),
    )


def _make_sc_deg(n, d, e):
    mesh = plsc.VectorSubcoreMesh(core_axis_name="c", subcore_axis_name="s")
    return pl.kernel(
        functools.partial(_sc_deg_body, n, d, e),
        out_type=(jax.ShapeDtypeStruct((NC, n, d), jnp.float32),),
        mesh=mesh,
        scratch_types=(
            pltpu.VMEM((5, KE), jnp.int32),
            pltpu.VMEM((5, KE), jnp.float32),
            pltpu.VMEM((3, KE, d), jnp.float32),
            pltpu.SemaphoreType.DMA((5,)),
            pltpu.SemaphoreType.DMA((3,)),
            pltpu.VMEM_SHARED((n, d), jnp.float32),
        ),
    )


def _mlp_body(relu, n, blk, p_ref, sw_ref, we_ref, be_ref, w1_ref, b1_ref,
              w2_ref, b2_ref, g_ref, bb_ref, o_ref, h2_scr, stats_scr):
    ph = pl.program_id(0)
    i = pl.program_id(1)

    @pl.when(ph == 0)
    def _():
        pp = p_ref[0] + p_ref[1]
        sw = sw_ref[0] + sw_ref[1]
        sumw = sw[:, 0:1]
        deg = sw[:, 1:2]
        aggr = pp + sumw * we_ref[...] + deg * be_ref[...]
        hid = lax.dot_general(aggr, w1_ref[...], (((1,), (0,)), ((), ())),
                              preferred_element_type=jnp.float32) + b1_ref[...]
        hid = jnp.maximum(hid, 0.0)
        h2 = lax.dot_general(hid, w2_ref[...], (((1,), (0,)), ((), ())),
                             preferred_element_type=jnp.float32) + b2_ref[...]
        h2_scr[pl.ds(i * blk, blk), :] = h2

        @pl.when(i == 0)
        def _():
            stats_scr[...] = jnp.zeros_like(stats_scr)

        stats_scr[0:1, :] += jnp.sum(h2, axis=0, keepdims=True)
        stats_scr[1:2, :] += jnp.sum(h2 * h2, axis=0, keepdims=True)

    @pl.when(ph == 1)
    def _():
        inv_n = 1.0 / n
        mean = stats_scr[0:1, :] * inv_n
        ex2 = stats_scr[1:2, :] * inv_n
        var = ex2 - mean * mean
        scale = g_ref[...] * lax.rsqrt(var + 1e-5)
        shift = bb_ref[...] - mean * scale
        o = h2_scr[pl.ds(i * blk, blk), :] * scale + shift
        if relu:
            o = jnp.maximum(o, 0.0)
        o_ref[...] = o


def _tc_layer(p, swdeg, params, relu, n, d, blk=2000):
    we = params['We'].reshape(1, d)
    be = params['be'].reshape(1, d)
    b1 = params['b1'].reshape(1, -1)
    b2 = params['b2'].reshape(1, -1)
    dh = params['W1'].shape[1]
    return pl.pallas_call(
        functools.partial(_mlp_body, relu, n, blk),
        grid=(2, n // blk),
        in_specs=[
            pl.BlockSpec((NC, blk, d), lambda p, i: (0, i, 0)),
            pl.BlockSpec((NC, blk, d), lambda p, i: (0, i, 0)),
            pl.BlockSpec((1, d), lambda p, i: (0, 0)),
            pl.BlockSpec((1, d), lambda p, i: (0, 0)),
            pl.BlockSpec((d, dh), lambda p, i: (0, 0)),
            pl.BlockSpec((1, dh), lambda p, i: (0, 0)),
            pl.BlockSpec((dh, d), lambda p, i: (0, 0)),
            pl.BlockSpec((1, d), lambda p, i: (0, 0)),
            pl.BlockSpec((1, d), lambda p, i: (0, 0)),
            pl.BlockSpec((1, d), lambda p, i: (0, 0)),
        ],
        out_specs=pl.BlockSpec((blk, d), lambda p, i: (i, 0)),
        out_shape=jax.ShapeDtypeStruct((n, d), jnp.float32),
        scratch_shapes=[
            pltpu.VMEM((n, d), jnp.float32),
            pltpu.VMEM((8, d), jnp.float32),
        ],
    )(p, swdeg, we, be, params['W1'], b1, params['W2'], b2,
      params['gamma'].reshape(1, d), params['beta'].reshape(1, d))


def kernel(x, edge_index, edge_weight, params):
    b, g, d = x.shape
    n = b * g
    e = edge_index.shape[1]
    h = x.reshape(n, d)
    ei = edge_index.reshape(2 * e)

    sc_main = _make_sc_main(n, d, e)
    sc_deg = _make_sc_deg(n, d, e)

    (partial,) = sc_main(h, ei)
    tok = (partial[0, 0, 0] * 0.0).astype(jnp.int32)
    (swdeg,) = sc_deg(ei + tok, edge_weight)

    num_layers = len(params)
    for l in range(num_layers):
        if l > 0:
            (partial,) = sc_main(h, ei)
        h = _tc_layer(partial, swdeg, params[l],
                      relu=(l < num_layers - 1), n=n, d=d)
    return h.reshape(b, g, d)
